# baseline (speedup 1.0000x reference)
import jax
import jax.numpy as jnp
import numpy as np
from jax import lax
from jax.experimental import pallas as pl
from jax.experimental.pallas import tpu as pltpu
from jax.experimental.pallas import tpu_sc as plsc

_MAP = np.array([99, 124, 119, 123, 242, 107, 111, 197, 48, 1, 103, 43, 254, 215, 171, 118, 202, 130, 201, 125, 250, 89, 71, 240, 173, 212, 162, 175, 156, 164, 114, 192, 183, 253, 147, 38, 54, 63, 247, 204, 52, 165, 229, 241, 113, 216, 49, 21, 4, 199, 35, 195, 24, 150, 5, 154, 7, 18, 128, 226, 235, 39, 178, 117, 9, 131, 44, 26, 27, 110, 90, 160, 82, 59, 214, 179, 41, 227, 47, 132, 83, 209, 0, 237, 32, 252, 177, 91, 106, 203, 190, 57, 74, 76, 88, 207, 208, 239, 170, 251, 67, 77, 51, 133, 69, 249, 2, 127, 80, 60, 159, 168, 81, 163, 64, 143, 146, 157, 56, 245, 188, 182, 218, 33, 16, 255, 243, 210, 205, 12, 19, 236, 95, 151, 68, 23, 196, 167, 126, 61, 100, 93, 25, 115, 96, 129, 79, 220, 34, 42, 144, 136, 70, 238, 184, 20, 222, 94, 11, 219, 224, 50, 58, 10, 73, 6, 36, 92, 194, 211, 172, 98, 145, 149, 228, 121, 231, 200, 55, 109, 141, 213, 78, 169, 108, 86, 244, 234, 101, 122, 174, 8, 186, 120, 37, 46, 28, 166, 180, 198, 232, 221, 116, 31, 75, 189, 139, 138, 112, 62, 181, 102, 72, 3, 246, 14, 97, 53, 87, 185, 134, 193, 29, 158, 225, 248, 152, 17, 105, 217, 142, 148, 155, 30, 135, 233, 206, 85, 40, 223, 140, 161, 137, 13, 191, 230, 66, 104, 65, 153, 45, 15, 176, 84, 187, 22], dtype=np.int32)

_N, _C = 16384, 256
_L = 16
_NW = 32
_BLK = 2048
_NBLK = _N // 8
_BLK_PER_W = _NBLK // _NW
_RB = 8
_NCHUNK = _BLK_PER_W // _RB
_CHW = _RB * _BLK

def _tile_off(x):
    return (x // 128) * 1024 + (x % 128)


def _koenig_groups():
    remaining = [(j % 16, int(_MAP[j]) % 16, j) for j in range(256)]
    groups = []
    for _ in range(16):
        adj = {}
        for a, b, j in remaining:
            adj.setdefault(a, []).append((b, j))
        match_l, match_r = {}, {}

        def try_aug(a, seen):
            for b, j in adj[a]:
                if b in seen:
                    continue
                seen.add(b)
                if b not in match_r or try_aug(match_r[b][0], seen):
                    match_l[a] = (b, j)
                    match_r[b] = (a, j)
                    return True
            return False

        for a in range(16):
            if not try_aug(a, set()):
                raise AssertionError("matching failed")
        chosen = {j for _, j in match_l.values()}
        groups.append(sorted(chosen))
        remaining = [e for e in remaining if e[2] not in chosen]
    return groups


_GROUPS = _koenig_groups()
_OFF_TAB = np.concatenate([
    np.array([_tile_off(int(_MAP[j])) for grp in _GROUPS for j in grp], np.int32),
    np.array([_tile_off(j) for grp in _GROUPS for j in grp], np.int32),
])


def _compute_chunk(in_v, out_v, off_v):
    ivecs = tuple(off_v[pl.ds(g * _L, _L)] for g in range(16))
    ovecs = tuple(off_v[pl.ds(256 + g * _L, _L)] for g in range(16))

    @plsc.parallel_loop(0, _RB, unroll=3, carry=(ivecs, ovecs))
    def block(b, carry):
        ivs, ovs = carry
        b0 = b * _BLK
        for s in range(8):
            sb = b0 + s * 128
            for g in range(16):
                gat = plsc.load_gather(in_v, [sb + ivs[g]])
                plsc.store_scatter(out_v, [sb + ovs[g]], gat)
        return carry


def _body(in_hbm, off_hbm, out_hbm,
          in_v0, in_v1, out_v0, out_v1, off_v, si0, si1, so0, so1):
    wid = lax.axis_index("s") * 2 + lax.axis_index("c")
    base = wid * _BLK_PER_W * _BLK
    pltpu.sync_copy(off_hbm, off_v)
    inb, outb = (in_v0, in_v1), (out_v0, out_v1)
    sin, sout = (si0, si1), (so0, so1)

    def in_slice(ci):
        return in_hbm.at[pl.ds(base + ci * _CHW, _CHW)]

    def out_slice(ci):
        return out_hbm.at[pl.ds(base + ci * _CHW, _CHW)]

    pltpu.async_copy(in_slice(0), inb[0], sin[0])
    pltpu.make_async_copy(in_slice(0), inb[0], sin[0]).wait()
    pltpu.async_copy(in_slice(1), inb[1], sin[1])
    _compute_chunk(inb[0], outb[0], off_v)
    pltpu.async_copy(outb[0], out_slice(0), sout[0])
    pltpu.make_async_copy(in_slice(1), inb[1], sin[1]).wait()
    pltpu.async_copy(in_slice(2), inb[0], sin[0])
    _compute_chunk(inb[1], outb[1], off_v)
    pltpu.async_copy(outb[1], out_slice(1), sout[1])

    def pair(k, carry):
        c0 = 2 * k
        pltpu.make_async_copy(outb[0], out_slice(c0 - 2), sout[0]).wait()
        pltpu.make_async_copy(in_slice(c0), inb[0], sin[0]).wait()
        pltpu.async_copy(in_slice(c0 + 1), inb[1], sin[1])
        _compute_chunk(inb[0], outb[0], off_v)
        pltpu.async_copy(outb[0], out_slice(c0), sout[0])
        pltpu.make_async_copy(outb[1], out_slice(c0 - 1), sout[1]).wait()
        pltpu.make_async_copy(in_slice(c0 + 1), inb[1], sin[1]).wait()
        nxt = jnp.minimum(c0 + 2, _NCHUNK - 1)
        pltpu.async_copy(in_slice(nxt), inb[0], sin[0])
        _compute_chunk(inb[1], outb[1], off_v)
        pltpu.async_copy(outb[1], out_slice(c0 + 1), sout[1])
        return carry

    lax.fori_loop(1, _NCHUNK // 2, pair, 0)
    pltpu.make_async_copy(in_slice(_NCHUNK - 1), inb[0], sin[0]).wait()
    pltpu.make_async_copy(outb[0], out_slice(_NCHUNK - 2), sout[0]).wait()
    pltpu.make_async_copy(outb[1], out_slice(_NCHUNK - 1), sout[1]).wait()


@jax.jit
def kernel(inputs):
    mesh = plsc.VectorSubcoreMesh(core_axis_name="c", subcore_axis_name="s",
                                  num_cores=2, num_subcores=16)
    f = pl.kernel(
        _body,
        out_type=jax.ShapeDtypeStruct((_N * _C,), jnp.float32),
        mesh=mesh,
        scratch_types=[
            pltpu.VMEM((_CHW,), jnp.float32),
            pltpu.VMEM((_CHW,), jnp.float32),
            pltpu.VMEM((_CHW,), jnp.float32),
            pltpu.VMEM((_CHW,), jnp.float32),
            pltpu.VMEM((2 * _C,), jnp.int32),
            pltpu.SemaphoreType.DMA,
            pltpu.SemaphoreType.DMA,
            pltpu.SemaphoreType.DMA,
            pltpu.SemaphoreType.DMA,
        ],
        compiler_params=pltpu.CompilerParams(needs_layout_passes=False),
    )
    tiled = inputs.reshape(_NBLK, 8, 2, 128).transpose(0, 2, 1, 3).reshape(_N * _C)
    y = f(tiled, jnp.asarray(_OFF_TAB))
    return y.reshape(_NBLK, 2, 8, 128).transpose(0, 2, 1, 3).reshape(_N, _C)


# speedup vs baseline: 1.3421x; 1.3421x over previous
import jax
import jax.numpy as jnp
import numpy as np
from jax import lax
from jax.experimental import pallas as pl
from jax.experimental.pallas import tpu as pltpu
from jax.experimental.pallas import tpu_sc as plsc

_MAP = np.array([99, 124, 119, 123, 242, 107, 111, 197, 48, 1, 103, 43, 254, 215, 171, 118, 202, 130, 201, 125, 250, 89, 71, 240, 173, 212, 162, 175, 156, 164, 114, 192, 183, 253, 147, 38, 54, 63, 247, 204, 52, 165, 229, 241, 113, 216, 49, 21, 4, 199, 35, 195, 24, 150, 5, 154, 7, 18, 128, 226, 235, 39, 178, 117, 9, 131, 44, 26, 27, 110, 90, 160, 82, 59, 214, 179, 41, 227, 47, 132, 83, 209, 0, 237, 32, 252, 177, 91, 106, 203, 190, 57, 74, 76, 88, 207, 208, 239, 170, 251, 67, 77, 51, 133, 69, 249, 2, 127, 80, 60, 159, 168, 81, 163, 64, 143, 146, 157, 56, 245, 188, 182, 218, 33, 16, 255, 243, 210, 205, 12, 19, 236, 95, 151, 68, 23, 196, 167, 126, 61, 100, 93, 25, 115, 96, 129, 79, 220, 34, 42, 144, 136, 70, 238, 184, 20, 222, 94, 11, 219, 224, 50, 58, 10, 73, 6, 36, 92, 194, 211, 172, 98, 145, 149, 228, 121, 231, 200, 55, 109, 141, 213, 78, 169, 108, 86, 244, 234, 101, 122, 174, 8, 186, 120, 37, 46, 28, 166, 180, 198, 232, 221, 116, 31, 75, 189, 139, 138, 112, 62, 181, 102, 72, 3, 246, 14, 97, 53, 87, 185, 134, 193, 29, 158, 225, 248, 152, 17, 105, 217, 142, 148, 155, 30, 135, 233, 206, 85, 40, 223, 140, 161, 137, 13, 191, 230, 66, 104, 65, 153, 45, 15, 176, 84, 187, 22], dtype=np.int32)

_N, _C = 16384, 256
_L = 16
_NW = 32
_BLK = 2048
_NBLK = _N // 8
_BLK_PER_W = _NBLK // _NW
_RB = 8
_NCHUNK = _BLK_PER_W // _RB
_CHW = _RB * _BLK

def _tile_off(x):
    return (x // 128) * 1024 + (x % 128)


def _koenig_groups():
    remaining = [(j % 16, int(_MAP[j]) % 16, j) for j in range(256)]
    groups = []
    for _ in range(16):
        adj = {}
        for a, b, j in remaining:
            adj.setdefault(a, []).append((b, j))
        match_l, match_r = {}, {}

        def try_aug(a, seen):
            for b, j in adj[a]:
                if b in seen:
                    continue
                seen.add(b)
                if b not in match_r or try_aug(match_r[b][0], seen):
                    match_l[a] = (b, j)
                    match_r[b] = (a, j)
                    return True
            return False

        for a in range(16):
            if not try_aug(a, set()):
                raise AssertionError("matching failed")
        chosen = {j for _, j in match_l.values()}
        groups.append(sorted(chosen))
        remaining = [e for e in remaining if e[2] not in chosen]
    return groups


_GROUPS = _koenig_groups()
_OFF_TAB = np.concatenate([
    np.array([_tile_off(int(_MAP[j])) for grp in _GROUPS for j in grp], np.int32),
    np.array([_tile_off(j) for grp in _GROUPS for j in grp], np.int32),
])


def _compute_chunk(in_v, out_v, off_v):
    ivecs = tuple(off_v[pl.ds(g * _L, _L)] for g in range(16))
    ovecs = tuple(off_v[pl.ds(256 + g * _L, _L)] for g in range(16))

    @plsc.parallel_loop(0, _RB, unroll=2, carry=(ivecs, ovecs))
    def block(b, carry):
        ivs, ovs = carry
        b0 = b * _BLK
        for s in range(8):
            sb = b0 + s * 128
            for g in range(16):
                gat = plsc.load_gather(in_v, [sb + ivs[g]])
                plsc.store_scatter(out_v, [sb + ovs[g]], gat)
        return carry


def _body(in_hbm, off_hbm, out_hbm,
          in_v0, in_v1, out_v0, out_v1, off_v, si0, si1, so0, so1):
    wid = lax.axis_index("s") * 2 + lax.axis_index("c")
    base = wid * _BLK_PER_W * _BLK
    pltpu.sync_copy(off_hbm, off_v)
    inb, outb = (in_v0, in_v1), (out_v0, out_v1)
    sin, sout = (si0, si1), (so0, so1)

    def in_slice(ci):
        return in_hbm.at[pl.ds(base + ci * _CHW, _CHW)]

    def out_slice(ci):
        return out_hbm.at[pl.ds(base + ci * _CHW, _CHW)]

    pltpu.async_copy(in_slice(0), inb[0], sin[0])
    pltpu.make_async_copy(in_slice(0), inb[0], sin[0]).wait()
    pltpu.async_copy(in_slice(1), inb[1], sin[1])
    _compute_chunk(inb[0], outb[0], off_v)
    pltpu.async_copy(outb[0], out_slice(0), sout[0])
    pltpu.make_async_copy(in_slice(1), inb[1], sin[1]).wait()
    pltpu.async_copy(in_slice(2), inb[0], sin[0])
    _compute_chunk(inb[1], outb[1], off_v)
    pltpu.async_copy(outb[1], out_slice(1), sout[1])

    def pair(k, carry):
        c0 = 2 * k
        pltpu.make_async_copy(outb[0], out_slice(c0 - 2), sout[0]).wait()
        pltpu.make_async_copy(in_slice(c0), inb[0], sin[0]).wait()
        pltpu.async_copy(in_slice(c0 + 1), inb[1], sin[1])
        _compute_chunk(inb[0], outb[0], off_v)
        pltpu.async_copy(outb[0], out_slice(c0), sout[0])
        pltpu.make_async_copy(outb[1], out_slice(c0 - 1), sout[1]).wait()
        pltpu.make_async_copy(in_slice(c0 + 1), inb[1], sin[1]).wait()
        nxt = jnp.minimum(c0 + 2, _NCHUNK - 1)
        pltpu.async_copy(in_slice(nxt), inb[0], sin[0])
        _compute_chunk(inb[1], outb[1], off_v)
        pltpu.async_copy(outb[1], out_slice(c0 + 1), sout[1])
        return carry

    lax.fori_loop(1, _NCHUNK // 2, pair, 0)
    pltpu.make_async_copy(in_slice(_NCHUNK - 1), inb[0], sin[0]).wait()
    pltpu.make_async_copy(outb[0], out_slice(_NCHUNK - 2), sout[0]).wait()
    pltpu.make_async_copy(outb[1], out_slice(_NCHUNK - 1), sout[1]).wait()


@jax.jit
def kernel(inputs):
    mesh = plsc.VectorSubcoreMesh(core_axis_name="c", subcore_axis_name="s",
                                  num_cores=2, num_subcores=16)
    f = pl.kernel(
        _body,
        out_type=jax.ShapeDtypeStruct((_N * _C,), jnp.float32),
        mesh=mesh,
        scratch_types=[
            pltpu.VMEM((_CHW,), jnp.float32),
            pltpu.VMEM((_CHW,), jnp.float32),
            pltpu.VMEM((_CHW,), jnp.float32),
            pltpu.VMEM((_CHW,), jnp.float32),
            pltpu.VMEM((2 * _C,), jnp.int32),
            pltpu.SemaphoreType.DMA,
            pltpu.SemaphoreType.DMA,
            pltpu.SemaphoreType.DMA,
            pltpu.SemaphoreType.DMA,
        ],
        compiler_params=pltpu.CompilerParams(needs_layout_passes=False),
    )
    tiled = inputs.reshape(_NBLK, 8, 2, 128).transpose(0, 2, 1, 3).reshape(_N * _C)
    y = f(tiled, jnp.asarray(_OFF_TAB))
    return y.reshape(_NBLK, 2, 8, 128).transpose(0, 2, 1, 3).reshape(_N, _C)
